# speedup vs baseline: 5.1976x; 5.1976x over previous
import functools

import jax
import jax.numpy as jnp
from jax import lax
from jax.experimental import pallas as pl
from jax.experimental.pallas import tpu as pltpu
from jax.experimental.pallas import tpu_sc as plsc

D = 128
B = 320000


@jax.jit
def _gather(x, idx):
    info = plsc.get_sparse_core_info()
    nw = info.num_cores * info.num_subcores
    b_per_w = B // nw
    chunk = 400
    n_chunks = b_per_w // chunk
    mesh = plsc.VectorSubcoreMesh(core_axis_name="c", subcore_axis_name="s")

    @functools.partial(
        pl.kernel,
        mesh=mesh,
        out_type=jax.ShapeDtypeStruct((B, D), jnp.float32),
        scratch_types=[
            pltpu.VMEM((chunk,), jnp.int32),
            pltpu.VMEM((chunk, D), jnp.float32),
            pltpu.SemaphoreType.DMA,
        ],
    )
    def k(x_hbm, idx_hbm, out_hbm, idx_v, rows_v, sem):
        wid = lax.axis_index("s") * info.num_cores + lax.axis_index("c")
        base = wid * b_per_w

        def body(c, _):
            off = base + c * chunk
            pltpu.sync_copy(idx_hbm.at[pl.ds(off, chunk)], idx_v)
            pltpu.async_copy(x_hbm.at[idx_v], rows_v, sem).wait()
            pltpu.sync_copy(rows_v, out_hbm.at[pl.ds(off, chunk)])
            return ()

        lax.fori_loop(0, n_chunks, body, ())

    return k(x, idx)


def kernel(x, neighbor_indices):
    return _gather(x, neighbor_indices.astype(jnp.int32))


# speedup vs baseline: 5.9019x; 1.1355x over previous
import functools

import jax
import jax.numpy as jnp
from jax import lax
from jax.experimental import pallas as pl
from jax.experimental.pallas import tpu as pltpu
from jax.experimental.pallas import tpu_sc as plsc

D = 128
B = 320000


@jax.jit
def _gather(x, idx):
    info = plsc.get_sparse_core_info()
    nw = info.num_cores * info.num_subcores
    b_per_w = B // nw
    chunk = 200
    n_chunks = b_per_w // chunk
    mesh = plsc.VectorSubcoreMesh(core_axis_name="c", subcore_axis_name="s")

    @functools.partial(
        pl.kernel,
        mesh=mesh,
        out_type=jax.ShapeDtypeStruct((B, D), jnp.float32),
        scratch_types=[
            pltpu.VMEM((chunk,), jnp.int32),
            pltpu.VMEM((chunk,), jnp.int32),
            pltpu.VMEM((chunk, D), jnp.float32),
            pltpu.VMEM((chunk, D), jnp.float32),
            pltpu.SemaphoreType.DMA,
            pltpu.SemaphoreType.DMA,
        ],
    )
    def k(x_hbm, idx_hbm, out_hbm, idx0, idx1, rows0, rows1, g0, g1):
        wid = lax.axis_index("s") * info.num_cores + lax.axis_index("c")
        base = wid * b_per_w
        idxs = (idx0, idx1)
        rows = (rows0, rows1)
        sems = (g0, g1)
        for b in range(2):
            pltpu.sync_copy(idx_hbm.at[pl.ds(base + b * chunk, chunk)], idxs[b])
            pltpu.async_copy(x_hbm.at[idxs[b]], rows[b], sems[b])

        def pair(p, _):
            c = 2 * p
            for b in range(2):
                cb = c + b
                pltpu.make_async_copy(
                    x_hbm.at[idxs[b]], rows[b], sems[b]).wait()
                pltpu.sync_copy(
                    rows[b], out_hbm.at[pl.ds(base + cb * chunk, chunk)])
                pltpu.sync_copy(
                    idx_hbm.at[pl.ds(base + (cb + 2) * chunk, chunk)], idxs[b])
                pltpu.async_copy(x_hbm.at[idxs[b]], rows[b], sems[b])
            return ()

        lax.fori_loop(0, n_chunks // 2 - 1, pair, ())
        for b in range(2):
            cb = n_chunks - 2 + b
            pltpu.make_async_copy(x_hbm.at[idxs[b]], rows[b], sems[b]).wait()
            pltpu.sync_copy(
                rows[b], out_hbm.at[pl.ds(base + cb * chunk, chunk)])

    return k(x, idx)


def kernel(x, neighbor_indices):
    return _gather(x, neighbor_indices.astype(jnp.int32))


# speedup vs baseline: 7.5385x; 1.2773x over previous
import functools

import jax
import jax.numpy as jnp
from jax import lax
from jax.experimental import pallas as pl
from jax.experimental.pallas import tpu as pltpu
from jax.experimental.pallas import tpu_sc as plsc

D = 128
B = 320000
N_ROWS = 10000


@jax.jit
def _gather(x, idx):
    info = plsc.get_sparse_core_info()
    nw = info.num_cores * info.num_subcores
    b_per_w = B // nw
    chunk = 192
    n_full = b_per_w // chunk
    tail = b_per_w - n_full * chunk
    n_chunks = n_full + 1
    n_pairs = (n_full - 2) // 2
    tail_start = 2 * n_pairs
    mesh = plsc.VectorSubcoreMesh(core_axis_name="c", subcore_axis_name="s")

    @functools.partial(
        pl.kernel,
        mesh=mesh,
        out_type=jax.ShapeDtypeStruct((B, D), jnp.float32),
        scratch_types=[
            pltpu.VMEM_SHARED((N_ROWS, D), jnp.float32),
            pltpu.VMEM((chunk,), jnp.int32),
            pltpu.VMEM((chunk,), jnp.int32),
            pltpu.VMEM((chunk, D), jnp.float32),
            pltpu.VMEM((chunk, D), jnp.float32),
            pltpu.SemaphoreType.DMA,
            pltpu.SemaphoreType.DMA,
        ],
    )
    def k(x_hbm, idx_hbm, out_hbm, x_sp, idx0, idx1, rows0, rows1, g0, g1):
        sid = lax.axis_index("s")
        wid = sid * info.num_cores + lax.axis_index("c")
        base = wid * b_per_w
        idxs = (idx0, idx1)
        rows = (rows0, rows1)
        sems = (g0, g1)

        @pl.when(sid < 10)
        def _stage():
            r0 = sid * 1000
            pltpu.sync_copy(x_hbm.at[pl.ds(r0, 1000)], x_sp.at[pl.ds(r0, 1000)])

        plsc.subcore_barrier()

        def load_and_fire(cb, b, size):
            pltpu.sync_copy(
                idx_hbm.at[pl.ds(base + cb * chunk, size)],
                idxs[b].at[pl.ds(0, size)])
            pltpu.async_copy(
                x_sp.at[idxs[b].at[pl.ds(0, size)]],
                rows[b].at[pl.ds(0, size)], sems[b])

        def finish(cb, b, size):
            pltpu.make_async_copy(
                x_sp.at[idxs[b].at[pl.ds(0, size)]],
                rows[b].at[pl.ds(0, size)], sems[b]).wait()
            pltpu.sync_copy(
                rows[b].at[pl.ds(0, size)],
                out_hbm.at[pl.ds(base + cb * chunk, size)])

        for b in range(2):
            load_and_fire(b, b, chunk)

        def pair(p, _):
            c = 2 * p
            for b in range(2):
                finish(c + b, b, chunk)
                load_and_fire(c + b + 2, b, chunk)
            return ()

        lax.fori_loop(0, n_pairs, pair, ())

        for cb in range(tail_start, n_chunks):
            b = cb % 2
            size = chunk if cb < n_full else tail
            finish(cb, b, size)
            if cb + 2 < n_chunks:
                nsize = chunk if cb + 2 < n_full else tail
                load_and_fire(cb + 2, b, nsize)

    return k(x, idx)


def kernel(x, neighbor_indices):
    return _gather(x, neighbor_indices.astype(jnp.int32))


# speedup vs baseline: 8.9946x; 1.1931x over previous
import functools

import jax
import jax.numpy as jnp
from jax import lax
from jax.experimental import pallas as pl
from jax.experimental.pallas import tpu as pltpu
from jax.experimental.pallas import tpu_sc as plsc

D = 128
B = 320000
N_ROWS = 10000


@jax.jit
def _gather(x, idx):
    info = plsc.get_sparse_core_info()
    nw = info.num_cores * info.num_subcores
    b_per_w = B // nw
    chunk = 128
    n_full = b_per_w // chunk
    tail = b_per_w - n_full * chunk
    n_chunks = n_full + 1
    mesh = plsc.VectorSubcoreMesh(core_axis_name="c", subcore_axis_name="s")

    n_triples = (n_full - 1 - 3 - 2) // 3
    tail_start = 3 * n_triples

    @functools.partial(
        pl.kernel,
        mesh=mesh,
        out_type=jax.ShapeDtypeStruct((B, D), jnp.float32),
        scratch_types=[
            pltpu.VMEM_SHARED((N_ROWS, D), jnp.float32),
            pltpu.VMEM((chunk,), jnp.int32),
            pltpu.VMEM((chunk,), jnp.int32),
            pltpu.VMEM((chunk,), jnp.int32),
            pltpu.VMEM((chunk, D), jnp.float32),
            pltpu.VMEM((chunk, D), jnp.float32),
            pltpu.VMEM((chunk, D), jnp.float32),
            pltpu.SemaphoreType.DMA,
            pltpu.SemaphoreType.DMA,
            pltpu.SemaphoreType.DMA,
            pltpu.SemaphoreType.DMA,
            pltpu.SemaphoreType.DMA,
            pltpu.SemaphoreType.DMA,
            pltpu.SemaphoreType.DMA,
            pltpu.SemaphoreType.DMA,
            pltpu.SemaphoreType.DMA,
        ],
    )
    def k(x_hbm, idx_hbm, out_hbm, x_sp,
          i0, i1, i2, r0, r1, r2,
          gi0, gi1, gi2, g0, g1, g2, w0, w1, w2):
        sid = lax.axis_index("s")
        wid = sid * info.num_cores + lax.axis_index("c")
        base = wid * b_per_w
        idxs = (i0, i1, i2)
        rows = (r0, r1, r2)
        gis = (gi0, gi1, gi2)
        gs = (g0, g1, g2)
        ws = (w0, w1, w2)

        @pl.when(sid < 10)
        def _stage():
            r = sid * 1000
            pltpu.sync_copy(x_hbm.at[pl.ds(r, 1000)], x_sp.at[pl.ds(r, 1000)])

        plsc.subcore_barrier()

        def sz(c):
            return chunk if c < n_full else tail

        def fire_idx(c, b, size):
            pltpu.async_copy(
                idx_hbm.at[pl.ds(base + c * chunk, size)],
                idxs[b].at[pl.ds(0, size)], gis[b])

        def wait_idx(c, b, size):
            pltpu.make_async_copy(
                idx_hbm.at[pl.ds(base + c * chunk, size)],
                idxs[b].at[pl.ds(0, size)], gis[b]).wait()

        def fire_gather(c, b, size):
            wait_idx(c, b, size)
            pltpu.async_copy(
                x_sp.at[idxs[b].at[pl.ds(0, size)]],
                rows[b].at[pl.ds(0, size)], gs[b])

        def wait_gather(c, b, size):
            pltpu.make_async_copy(
                x_sp.at[idxs[b].at[pl.ds(0, size)]],
                rows[b].at[pl.ds(0, size)], gs[b]).wait()

        def fire_wb(c, b, size):
            pltpu.async_copy(
                rows[b].at[pl.ds(0, size)],
                out_hbm.at[pl.ds(base + c * chunk, size)], ws[b])

        def wait_wb(c, b, size):
            pltpu.make_async_copy(
                rows[b].at[pl.ds(0, size)],
                out_hbm.at[pl.ds(base + c * chunk, size)], ws[b]).wait()

        for c in range(3):
            fire_idx(c, c, chunk)
        for c in range(2):
            fire_gather(c, c, chunk)

        wait_gather(0, 0, chunk)
        fire_idx(3, 0, chunk)
        fire_wb(0, 0, chunk)
        fire_gather(2, 2, chunk)

        def triple(p, _):
            s = 3 * p + 1
            for q in range(3):
                sq = s + q
                bq = (1 + q) % 3
                wait_gather(sq, bq, chunk)
                fire_idx(sq + 3, bq, chunk)
                fire_wb(sq, bq, chunk)
                wait_wb(sq - 1, q % 3, chunk)
                fire_gather(sq + 2, q % 3, chunk)
            return ()

        lax.fori_loop(0, n_triples, triple, ())

        for s in range(3 * n_triples + 1, n_chunks):
            wait_gather(s, s % 3, sz(s))
            if s + 3 < n_chunks:
                fire_idx(s + 3, (s + 3) % 3, sz(s + 3))
            fire_wb(s, s % 3, sz(s))
            wait_wb(s - 1, (s - 1) % 3, sz(s - 1))
            if s + 2 < n_chunks:
                fire_gather(s + 2, (s + 2) % 3, sz(s + 2))
        wait_wb(n_chunks - 1, (n_chunks - 1) % 3, sz(n_chunks - 1))

    return k(x, idx)


def kernel(x, neighbor_indices):
    return _gather(x, neighbor_indices.astype(jnp.int32))


# speedup vs baseline: 9.5557x; 1.0624x over previous
import functools

import jax
import jax.numpy as jnp
from jax import lax
from jax.experimental import pallas as pl
from jax.experimental.pallas import tpu as pltpu
from jax.experimental.pallas import tpu_sc as plsc

D = 128
B = 320000
N_ROWS = 10000


@jax.jit
def _gather(x, idx):
    info = plsc.get_sparse_core_info()
    nw = info.num_cores * info.num_subcores
    b_per_w = B // nw
    chunk = 72
    n_full = b_per_w // chunk
    tail = b_per_w - n_full * chunk
    n_chunks = n_full + 1
    R = 5
    LAG = 3
    HBM_CHUNKS = 7
    n_quads = 28
    while 7 + R * n_quads - 1 + R >= n_full:
        n_quads -= 1
    loop_end = 7 + R * n_quads
    mesh = plsc.VectorSubcoreMesh(core_axis_name="c", subcore_axis_name="s")

    @functools.partial(
        pl.kernel,
        mesh=mesh,
        out_type=jax.ShapeDtypeStruct((B, D), jnp.float32),
        scratch_types=[
            pltpu.VMEM_SHARED((N_ROWS, D), jnp.float32),
            pltpu.VMEM((chunk,), jnp.int32),
            pltpu.VMEM((chunk,), jnp.int32),
            pltpu.VMEM((chunk,), jnp.int32),
            pltpu.VMEM((chunk,), jnp.int32),
            pltpu.VMEM((chunk,), jnp.int32),
            pltpu.VMEM((chunk, D), jnp.float32),
            pltpu.VMEM((chunk, D), jnp.float32),
            pltpu.VMEM((chunk, D), jnp.float32),
            pltpu.VMEM((chunk, D), jnp.float32),
            pltpu.VMEM((chunk, D), jnp.float32),
            pltpu.SemaphoreType.DMA,
            pltpu.SemaphoreType.DMA,
            pltpu.SemaphoreType.DMA,
            pltpu.SemaphoreType.DMA,
            pltpu.SemaphoreType.DMA,
            pltpu.SemaphoreType.DMA,
            pltpu.SemaphoreType.DMA,
            pltpu.SemaphoreType.DMA,
            pltpu.SemaphoreType.DMA,
            pltpu.SemaphoreType.DMA,
            pltpu.SemaphoreType.DMA,
            pltpu.SemaphoreType.DMA,
            pltpu.SemaphoreType.DMA,
            pltpu.SemaphoreType.DMA,
            pltpu.SemaphoreType.DMA,
            pltpu.SemaphoreType.DMA,
        ],
    )
    def k(x_hbm, idx_hbm, out_hbm, x_sp,
          i0, i1, i2, i3, i4, r0, r1, r2, r3, r4,
          gi0, gi1, gi2, gi3, gi4, g0, g1, g2, g3, g4,
          w0, w1, w2, w3, w4, ssem):
        sid = lax.axis_index("s")
        wid = sid * info.num_cores + lax.axis_index("c")
        base = wid * b_per_w
        idxs = (i0, i1, i2, i3, i4)
        rows = (r0, r1, r2, r3, r4)
        gis = (gi0, gi1, gi2, gi3, gi4)
        gs = (g0, g1, g2, g3, g4)
        ws = (w0, w1, w2, w3, w4)

        @pl.when(sid < 10)
        def _stage():
            r = sid * 1000
            pltpu.async_copy(
                x_hbm.at[pl.ds(r, 1000)], x_sp.at[pl.ds(r, 1000)], ssem)

        def sz(c):
            return chunk if c < n_full else tail

        def fire_idx(c, b, size):
            pltpu.async_copy(
                idx_hbm.at[pl.ds(base + c * chunk, size)],
                idxs[b].at[pl.ds(0, size)], gis[b])

        def wait_idx(c, b, size):
            pltpu.make_async_copy(
                idx_hbm.at[pl.ds(base + c * chunk, size)],
                idxs[b].at[pl.ds(0, size)], gis[b]).wait()

        def fire_gather(src_ref, c, b, size):
            wait_idx(c, b, size)
            pltpu.async_copy(
                src_ref.at[idxs[b].at[pl.ds(0, size)]],
                rows[b].at[pl.ds(0, size)], gs[b])

        def wait_gather(src_ref, c, b, size):
            pltpu.make_async_copy(
                src_ref.at[idxs[b].at[pl.ds(0, size)]],
                rows[b].at[pl.ds(0, size)], gs[b]).wait()

        def fire_wb(c, b, size):
            pltpu.async_copy(
                rows[b].at[pl.ds(0, size)],
                out_hbm.at[pl.ds(base + c * chunk, size)], ws[b])

        def wait_wb(c, b, size):
            pltpu.make_async_copy(
                rows[b].at[pl.ds(0, size)],
                out_hbm.at[pl.ds(base + c * chunk, size)], ws[b]).wait()

        def src_of(c):
            return x_hbm if c < HBM_CHUNKS else x_sp

        def static_step(s):
            wait_gather(src_of(s), s, s % R, sz(s))
            if s + R < n_chunks:
                fire_idx(s + R, (s + R) % R, sz(s + R))
            fire_wb(s, s % R, sz(s))
            if s >= LAG:
                wait_wb(s - LAG, (s - LAG) % R, sz(s - LAG))
            if s + 2 < n_chunks:
                fire_gather(src_of(s + 2), s + 2, (s + 2) % R, sz(s + 2))

        for c in range(R):
            fire_idx(c, c, chunk)
        for c in range(2):
            fire_gather(x_hbm, c, c, chunk)

        for s in range(5):
            static_step(s)

        @pl.when(sid < 10)
        def _stage_done():
            r = sid * 1000
            pltpu.make_async_copy(
                x_hbm.at[pl.ds(r, 1000)], x_sp.at[pl.ds(r, 1000)], ssem).wait()

        plsc.subcore_barrier()

        for s in range(5, 7):
            static_step(s)

        def quad(p, _):
            s = R * p + 7
            for q in range(R):
                sq = s + q
                bq = (7 + q) % R
                wait_gather(x_sp, sq, bq, chunk)
                fire_idx(sq + R, bq, chunk)
                fire_wb(sq, bq, chunk)
                wait_wb(sq - LAG, (7 + q - LAG) % R, chunk)
                fire_gather(x_sp, sq + 2, (7 + q + 2) % R, chunk)
            return ()

        lax.fori_loop(0, n_quads, quad, ())

        for s in range(loop_end, n_chunks):
            static_step(s)
        for c in range(n_chunks - LAG, n_chunks):
            wait_wb(c, c % R, sz(c))

    return k(x, idx)


def kernel(x, neighbor_indices):
    return _gather(x, neighbor_indices.astype(jnp.int32))
